# baseline (speedup 1.0000x reference)
import functools

import jax
import jax.numpy as jnp
from jax import lax
from jax.experimental import pallas as pl
from jax.experimental.pallas import tpu as pltpu
import jax.experimental.pallas.tpu_sc as plsc

E = 8
TOPK = 2
D = 2048
F = 1024
T = 4096
BLK = 256
NBLK = (2 * T + E * (BLK - 1)) // BLK + 1
P = NBLK * BLK
CSB = 256
NW = 32
TPT = T // NW
F32 = jnp.float32
I32 = jnp.int32


def _k1_body(x_ref, wr_ref, logits_ref, cosl_ref, l2l_ref, pos_ref, rw_ref,
             sel_ref, bexp_ref):
    x = x_ref[...]
    wr = wr_ref[...]
    logits = jnp.dot(x, wr, preferred_element_type=F32)
    logits_ref[...] = logits

    hi = lax.Precision.HIGHEST
    dn0 = (((0,), (0,)), ((), ()))
    eye = (lax.broadcasted_iota(I32, (E, E), 0)
           == lax.broadcasted_iota(I32, (E, E), 1)).astype(F32)
    gram = lax.dot_general(logits, logits, dn0, precision=hi)
    colsq = jnp.sum(gram * eye, axis=0, keepdims=True)
    cn = jnp.sqrt(colsq)
    denom = jnp.maximum(lax.dot_general(cn, cn, dn0, precision=hi), 1e-8)
    cosl_ref[...] = gram / denom
    sq_i = lax.dot_general(eye, colsq, (((1,), (1,)), ((), ())),
                           precision=hi)
    l2l_ref[...] = jnp.sqrt(jnp.maximum(sq_i + colsq - 2.0 * gram, 0.0))

    mx = jnp.max(logits, axis=1, keepdims=True)
    el = jnp.exp(logits - mx)
    probs = el / jnp.sum(el, axis=1, keepdims=True)
    iot = lax.broadcasted_iota(I32, (T, E), 1)
    m1 = jnp.max(probs, axis=1, keepdims=True)
    a1 = jnp.min(jnp.where(probs == m1, iot, E), axis=1, keepdims=True)
    probs2 = jnp.where(iot == a1, -jnp.inf, probs)
    m2 = jnp.max(probs2, axis=1, keepdims=True)
    a2 = jnp.min(jnp.where(probs2 == m2, iot, E), axis=1, keepdims=True)
    rw_ref[:, 0:1] = m1
    rw_ref[:, 1:2] = m2
    sel_ref[:, 0:1] = a1
    sel_ref[:, 1:2] = a2

    c0 = (iot == a1).astype(F32)
    c1 = (iot == a2).astype(F32)
    ct = c0 + c1
    tri = (lax.broadcasted_iota(I32, (CSB, CSB), 1)
           < lax.broadcasted_iota(I32, (CSB, CSB), 0)).astype(F32)
    run = jnp.zeros((1, E), F32)
    parts = []
    for bi in range(T // CSB):
        blk = lax.slice(ct, (bi * CSB, 0), ((bi + 1) * CSB, E))
        parts.append(jnp.dot(tri, blk, preferred_element_type=F32,
                             precision=hi) + run)
        run = run + jnp.sum(blk, axis=0, keepdims=True)
    base = jnp.concatenate(parts, axis=0)
    counts = run
    pcnt = jnp.ceil(counts / BLK) * BLK
    u8 = (lax.broadcasted_iota(I32, (E, E), 0)
          < lax.broadcasted_iota(I32, (E, E), 1)).astype(F32)
    start = jnp.dot(pcnt, u8, preferred_element_type=F32,
                    precision=hi)
    rank0 = jnp.sum(base * c0, axis=1, keepdims=True)
    rank1 = jnp.sum(base * c1, axis=1, keepdims=True)
    s0 = jnp.sum(start * c0, axis=1, keepdims=True)
    s1 = jnp.sum(start * c1, axis=1, keepdims=True)
    pos_ref[:, 0:1] = (s0 + rank0).astype(I32)
    pos_ref[:, 1:2] = (s1 + rank1).astype(I32)

    end_p = start + pcnt
    qcol = (lax.broadcasted_iota(I32, (NBLK, 1), 0) * BLK).astype(F32)
    ge = (qcol >= end_p).astype(F32)
    bexp_ref[...] = jnp.minimum(
        jnp.sum(ge, axis=1, keepdims=True), E - 1).astype(I32)


def _run_k1(x, w_router):
    return pl.pallas_call(
        _k1_body,
        out_shape=[
            jax.ShapeDtypeStruct((T, E), F32),
            jax.ShapeDtypeStruct((E, E), F32),
            jax.ShapeDtypeStruct((E, E), F32),
            jax.ShapeDtypeStruct((T, TOPK), I32),
            jax.ShapeDtypeStruct((T, TOPK), F32),
            jax.ShapeDtypeStruct((T, TOPK), I32),
            jax.ShapeDtypeStruct((NBLK, 1), I32),
        ],
        compiler_params=pltpu.CompilerParams(
            vmem_limit_bytes=100 * 1024 * 1024),
    )(x, w_router)


def _k2_body(x_hbm, pos0_hbm, pos1_hbm, xs_hbm, rowbuf, idx0, idx1, sem):
    wid = lax.axis_index("s") * 2 + lax.axis_index("c")
    pltpu.sync_copy(pos0_hbm.at[pl.ds(wid * (TPT // 16), TPT // 16)], idx0)
    pltpu.sync_copy(pos1_hbm.at[pl.ds(wid * (TPT // 16), TPT // 16)], idx1)
    base = wid * TPT
    for c in range(TPT // 16):
        pltpu.sync_copy(x_hbm.at[pl.ds(base + c * 16, 16)], rowbuf)
        pltpu.async_copy(rowbuf, xs_hbm.at[idx0.at[c]], sem).wait()
        pltpu.async_copy(rowbuf, xs_hbm.at[idx1.at[c]], sem).wait()


def _run_k2(x, pos0, pos1):
    mesh = plsc.VectorSubcoreMesh(core_axis_name="c", subcore_axis_name="s")
    return pl.kernel(
        _k2_body,
        out_type=jax.ShapeDtypeStruct((P, D), F32),
        mesh=mesh,
        scratch_types=[
            pltpu.VMEM((16, D), F32),
            pltpu.VMEM((TPT // 16, 16), I32),
            pltpu.VMEM((TPT // 16, 16), I32),
            pltpu.SemaphoreType.DMA,
        ],
    )(x, pos0.reshape(T // 16, 16), pos1.reshape(T // 16, 16))


def _k3_body(bexp_ref, xs_ref, wg_ref, wu_ref, wd_ref, ys_ref):
    del bexp_ref
    xb = xs_ref[...].astype(jnp.bfloat16)
    wg = wg_ref[...].reshape(D, F)
    wu = wu_ref[...].reshape(D, F)
    wd = wd_ref[...].reshape(F, D)
    g = jnp.dot(xb, wg, preferred_element_type=F32)
    u = jnp.dot(xb, wu, preferred_element_type=F32)
    h = (g * (1.0 / (1.0 + jnp.exp(-g))) * u).astype(jnp.bfloat16)
    ys_ref[...] = jnp.dot(h, wd, preferred_element_type=F32)


def _run_k3(xs, bexp, wg, wu, wd):
    grid_spec = pltpu.PrefetchScalarGridSpec(
        num_scalar_prefetch=1,
        grid=(NBLK,),
        in_specs=[
            pl.BlockSpec((BLK, D), lambda m, be: (m, 0)),
            pl.BlockSpec((1, D, F), lambda m, be: (be[m], 0, 0)),
            pl.BlockSpec((1, D, F), lambda m, be: (be[m], 0, 0)),
            pl.BlockSpec((1, F, D), lambda m, be: (be[m], 0, 0)),
        ],
        out_specs=pl.BlockSpec((BLK, D), lambda m, be: (m, 0)),
    )
    return pl.pallas_call(
        _k3_body,
        grid_spec=grid_spec,
        out_shape=jax.ShapeDtypeStruct((P, D), F32),
        compiler_params=pltpu.CompilerParams(
            dimension_semantics=("arbitrary",),
            vmem_limit_bytes=110 * 1024 * 1024),
    )(bexp, xs, wg.astype(jnp.bfloat16), wu.astype(jnp.bfloat16),
      wd.astype(jnp.bfloat16))


def _k4_body(ys_hbm, pos0_hbm, pos1_hbm, y0_hbm, y1_hbm,
             rowbuf0, rowbuf1, idx0, idx1, sem0, sem1):
    wid = lax.axis_index("s") * 2 + lax.axis_index("c")
    pltpu.sync_copy(pos0_hbm.at[pl.ds(wid * (TPT // 16), TPT // 16)], idx0)
    pltpu.sync_copy(pos1_hbm.at[pl.ds(wid * (TPT // 16), TPT // 16)], idx1)
    base = wid * TPT
    for c in range(TPT // 16):
        pltpu.async_copy(ys_hbm.at[idx0.at[c]], rowbuf0, sem0).wait()
        pltpu.sync_copy(rowbuf0, y0_hbm.at[pl.ds(base + c * 16, 16)])
        pltpu.async_copy(ys_hbm.at[idx1.at[c]], rowbuf1, sem1).wait()
        pltpu.sync_copy(rowbuf1, y1_hbm.at[pl.ds(base + c * 16, 16)])


def _run_k4(ys, pos0, pos1):
    mesh = plsc.VectorSubcoreMesh(core_axis_name="c", subcore_axis_name="s")
    return pl.kernel(
        _k4_body,
        out_type=[jax.ShapeDtypeStruct((T, D), F32),
                  jax.ShapeDtypeStruct((T, D), F32)],
        mesh=mesh,
        scratch_types=[
            pltpu.VMEM((16, D), F32),
            pltpu.VMEM((16, D), F32),
            pltpu.VMEM((TPT // 16, 16), I32),
            pltpu.VMEM((TPT // 16, 16), I32),
            pltpu.SemaphoreType.DMA,
            pltpu.SemaphoreType.DMA,
        ],
    )(ys, pos0.reshape(T // 16, 16), pos1.reshape(T // 16, 16))


K5B = 512
K5N = T // K5B


def _k5_body(y0_ref, y1_ref, rw0_ref, rw1_ref, sel0_ref, sel1_ref,
             final_ref, cos_ref, l2o_ref, gacc, sacc):
    m = pl.program_id(0)

    @pl.when(m == 0)
    def _():
        gacc[...] = jnp.zeros((E, E), F32)
        sacc[...] = jnp.zeros((1, E), F32)

    rw0 = rw0_ref[...].reshape(K5B, 1)
    rw1 = rw1_ref[...].reshape(K5B, 1)
    y0 = y0_ref[...] * rw0
    y1 = y1_ref[...] * rw1
    final_ref[...] = y0 + y1

    p = jnp.sum(y0 * y1, axis=1, keepdims=True)
    d0 = jnp.sum(y0 * y0, axis=1, keepdims=True)
    d1 = jnp.sum(y1 * y1, axis=1, keepdims=True)
    s0 = jnp.sum(y0, axis=1, keepdims=True)
    s1 = jnp.sum(y1, axis=1, keepdims=True)
    iot = lax.broadcasted_iota(I32, (K5B, E), 1)
    oh0 = (sel0_ref[...].reshape(K5B, 1) == iot).astype(F32)
    oh1 = (sel1_ref[...].reshape(K5B, 1) == iot).astype(F32)

    dn = (((0,), (0,)), ((), ()))
    hi = lax.Precision.HIGHEST
    spart = (lax.dot_general(s0, oh0, dn, precision=hi)
             + lax.dot_general(s1, oh1, dn, precision=hi))
    dpart = (lax.dot_general(d0, oh0, dn, precision=hi)
             + lax.dot_general(d1, oh1, dn, precision=hi))
    m01 = lax.dot_general(oh0, p * oh1, dn, precision=hi)
    m10 = lax.dot_general(oh1, p * oh0, dn, precision=hi)
    eye = (lax.broadcasted_iota(I32, (E, E), 0)
           == lax.broadcasted_iota(I32, (E, E), 1)).astype(F32)
    gacc[...] += m01 + m10 + eye * dpart
    sacc[...] += spart

    @pl.when(m == K5N - 1)
    def _():
        n_tot = float(T * D)
        g = gacc[...]
        s = sacc[...]
        c = g - lax.dot_general(s, s, dn, precision=hi) / n_tot
        cd = jnp.sum(c * eye, axis=0, keepdims=True)
        den = jnp.maximum(jnp.sqrt(jnp.maximum(cd, 0.0)), 1e-12)
        denom2 = lax.dot_general(den, den, dn, precision=hi)
        cos = c / denom2
        cos_ref[...] = cos
        sqn = jnp.sum(cos * eye, axis=0, keepdims=True)
        sq_i = lax.dot_general(eye, sqn, (((1,), (1,)), ((), ())),
                               precision=hi)
        l2o_ref[...] = jnp.sqrt(jnp.maximum(sq_i + sqn - 2.0 * cos, 0.0))


def _run_k5(y0, y1, rw, sel):
    rw0 = rw[:, 0].reshape(K5N, K5B, 1)
    rw1 = rw[:, 1].reshape(K5N, K5B, 1)
    sel0 = sel[:, 0].reshape(K5N, K5B, 1)
    sel1 = sel[:, 1].reshape(K5N, K5B, 1)
    return pl.pallas_call(
        _k5_body,
        grid=(K5N,),
        in_specs=[
            pl.BlockSpec((K5B, D), lambda m: (m, 0)),
            pl.BlockSpec((K5B, D), lambda m: (m, 0)),
            pl.BlockSpec((1, K5B, 1), lambda m: (m, 0, 0)),
            pl.BlockSpec((1, K5B, 1), lambda m: (m, 0, 0)),
            pl.BlockSpec((1, K5B, 1), lambda m: (m, 0, 0)),
            pl.BlockSpec((1, K5B, 1), lambda m: (m, 0, 0)),
        ],
        out_specs=[
            pl.BlockSpec((K5B, D), lambda m: (m, 0)),
            pl.BlockSpec((E, E), lambda m: (0, 0)),
            pl.BlockSpec((E, E), lambda m: (0, 0)),
        ],
        out_shape=[
            jax.ShapeDtypeStruct((T, D), F32),
            jax.ShapeDtypeStruct((E, E), F32),
            jax.ShapeDtypeStruct((E, E), F32),
        ],
        scratch_shapes=[pltpu.VMEM((E, E), F32), pltpu.VMEM((1, E), F32)],
        compiler_params=pltpu.CompilerParams(
            dimension_semantics=("arbitrary",),
            vmem_limit_bytes=100 * 1024 * 1024),
    )(y0, y1, rw0, rw1, sel0, sel1)


def kernel(hidden_states, W_router, Wg, Wu, Wd):
    b, s, d = hidden_states.shape
    x = hidden_states.reshape(-1, d)
    (logits, cosine_logits, l2_logits, pos, rw, sel, bexp) = _run_k1(
        x, W_router)
    pos0 = pos[:, 0]
    pos1 = pos[:, 1]
    xs = _run_k2(x, pos0, pos1)
    ys = _run_k3(xs, bexp.reshape(NBLK), Wg, Wu, Wd)
    y0, y1 = _run_k4(ys, pos0, pos1)
    final, cosine_out, l2_out = _run_k5(y0, y1, rw, sel)
    return (final.reshape(b, s, d), logits, cosine_logits, l2_logits,
            cosine_out, l2_out)


# speedup vs baseline: 1.1533x; 1.1533x over previous
import functools

import jax
import jax.numpy as jnp
from jax import lax
from jax.experimental import pallas as pl
from jax.experimental.pallas import tpu as pltpu
import jax.experimental.pallas.tpu_sc as plsc

E = 8
TOPK = 2
D = 2048
F = 1024
T = 4096
BLK = 256
NBLK = (2 * T + E * (BLK - 1)) // BLK + 1
P = NBLK * BLK
CSB = 256
NW = 32
TPT = T // NW
F32 = jnp.float32
I32 = jnp.int32


def _k1_body(x_ref, wr_ref, logits_ref, cosl_ref, l2l_ref, pos_ref, rw_ref,
             sel_ref, bexp_ref):
    x = x_ref[...]
    wr = wr_ref[...]
    logits = jnp.dot(x, wr, preferred_element_type=F32)
    logits_ref[...] = logits

    hi = lax.Precision.HIGHEST
    dn0 = (((0,), (0,)), ((), ()))
    eye = (lax.broadcasted_iota(I32, (E, E), 0)
           == lax.broadcasted_iota(I32, (E, E), 1)).astype(F32)
    gram = lax.dot_general(logits, logits, dn0, precision=hi)
    colsq = jnp.sum(gram * eye, axis=0, keepdims=True)
    cn = jnp.sqrt(colsq)
    denom = jnp.maximum(lax.dot_general(cn, cn, dn0, precision=hi), 1e-8)
    cosl_ref[...] = gram / denom
    sq_i = lax.dot_general(eye, colsq, (((1,), (1,)), ((), ())),
                           precision=hi)
    l2l_ref[...] = jnp.sqrt(jnp.maximum(sq_i + colsq - 2.0 * gram, 0.0))

    mx = jnp.max(logits, axis=1, keepdims=True)
    el = jnp.exp(logits - mx)
    probs = el / jnp.sum(el, axis=1, keepdims=True)
    iot = lax.broadcasted_iota(I32, (T, E), 1)
    m1 = jnp.max(probs, axis=1, keepdims=True)
    a1 = jnp.min(jnp.where(probs == m1, iot, E), axis=1, keepdims=True)
    probs2 = jnp.where(iot == a1, -jnp.inf, probs)
    m2 = jnp.max(probs2, axis=1, keepdims=True)
    a2 = jnp.min(jnp.where(probs2 == m2, iot, E), axis=1, keepdims=True)
    rw_ref[:, 0:1] = m1
    rw_ref[:, 1:2] = m2
    sel_ref[:, 0:1] = a1
    sel_ref[:, 1:2] = a2

    c0 = (iot == a1).astype(F32)
    c1 = (iot == a2).astype(F32)
    ct = c0 + c1
    tri = (lax.broadcasted_iota(I32, (CSB, CSB), 1)
           < lax.broadcasted_iota(I32, (CSB, CSB), 0)).astype(F32)
    run = jnp.zeros((1, E), F32)
    parts = []
    for bi in range(T // CSB):
        blk = lax.slice(ct, (bi * CSB, 0), ((bi + 1) * CSB, E))
        parts.append(jnp.dot(tri, blk, preferred_element_type=F32,
                             precision=hi) + run)
        run = run + jnp.sum(blk, axis=0, keepdims=True)
    base = jnp.concatenate(parts, axis=0)
    counts = run
    pcnt = jnp.ceil(counts / BLK) * BLK
    u8 = (lax.broadcasted_iota(I32, (E, E), 0)
          < lax.broadcasted_iota(I32, (E, E), 1)).astype(F32)
    start = jnp.dot(pcnt, u8, preferred_element_type=F32,
                    precision=hi)
    rank0 = jnp.sum(base * c0, axis=1, keepdims=True)
    rank1 = jnp.sum(base * c1, axis=1, keepdims=True)
    s0 = jnp.sum(start * c0, axis=1, keepdims=True)
    s1 = jnp.sum(start * c1, axis=1, keepdims=True)
    pos_ref[:, 0:1] = (s0 + rank0).astype(I32)
    pos_ref[:, 1:2] = (s1 + rank1).astype(I32)

    end_p = start + pcnt
    qcol = (lax.broadcasted_iota(I32, (NBLK, 1), 0) * BLK).astype(F32)
    ge = (qcol >= end_p).astype(F32)
    bexp_ref[...] = jnp.minimum(
        jnp.sum(ge, axis=1, keepdims=True), E - 1).astype(I32)


def _run_k1(x, w_router):
    return pl.pallas_call(
        _k1_body,
        out_shape=[
            jax.ShapeDtypeStruct((T, E), F32),
            jax.ShapeDtypeStruct((E, E), F32),
            jax.ShapeDtypeStruct((E, E), F32),
            jax.ShapeDtypeStruct((T, TOPK), I32),
            jax.ShapeDtypeStruct((T, TOPK), F32),
            jax.ShapeDtypeStruct((T, TOPK), I32),
            jax.ShapeDtypeStruct((NBLK, 1), I32),
        ],
        compiler_params=pltpu.CompilerParams(
            vmem_limit_bytes=100 * 1024 * 1024),
    )(x, w_router)


def _k2_body(x_hbm, pos0_hbm, pos1_hbm, xs_hbm, rowbuf, idx0, idx1, sem):
    wid = lax.axis_index("s") * 2 + lax.axis_index("c")
    pltpu.sync_copy(pos0_hbm.at[pl.ds(wid * (TPT // 16), TPT // 16)], idx0)
    pltpu.sync_copy(pos1_hbm.at[pl.ds(wid * (TPT // 16), TPT // 16)], idx1)
    base = wid * TPT
    for c in range(TPT // 16):
        pltpu.sync_copy(x_hbm.at[pl.ds(base + c * 16, 16)], rowbuf)
        d0 = pltpu.async_copy(rowbuf, xs_hbm.at[idx0.at[c]], sem)
        d1 = pltpu.async_copy(rowbuf, xs_hbm.at[idx1.at[c]], sem)
        d0.wait()
        d1.wait()


def _run_k2(x, pos0, pos1):
    mesh = plsc.VectorSubcoreMesh(core_axis_name="c", subcore_axis_name="s")
    return pl.kernel(
        _k2_body,
        out_type=jax.ShapeDtypeStruct((P, D), F32),
        mesh=mesh,
        scratch_types=[
            pltpu.VMEM((16, D), F32),
            pltpu.VMEM((TPT // 16, 16), I32),
            pltpu.VMEM((TPT // 16, 16), I32),
            pltpu.SemaphoreType.DMA,
        ],
    )(x, pos0.reshape(T // 16, 16), pos1.reshape(T // 16, 16))


def _k3_body(bexp_ref, xs_ref, wg_ref, wu_ref, wd_ref, ys_ref):
    del bexp_ref
    xb = xs_ref[...]
    wg = wg_ref[...].reshape(D, F)
    wu = wu_ref[...].reshape(D, F)
    wd = wd_ref[...].reshape(F, D)
    g = jnp.dot(xb, wg, preferred_element_type=F32)
    u = jnp.dot(xb, wu, preferred_element_type=F32)
    h = g * (1.0 / (1.0 + jnp.exp(-g))) * u
    ys_ref[...] = jnp.dot(h, wd, preferred_element_type=F32)


def _run_k3(xs, bexp, wg, wu, wd):
    grid_spec = pltpu.PrefetchScalarGridSpec(
        num_scalar_prefetch=1,
        grid=(NBLK,),
        in_specs=[
            pl.BlockSpec((BLK, D), lambda m, be: (m, 0)),
            pl.BlockSpec((1, D, F), lambda m, be: (be[m], 0, 0)),
            pl.BlockSpec((1, D, F), lambda m, be: (be[m], 0, 0)),
            pl.BlockSpec((1, F, D), lambda m, be: (be[m], 0, 0)),
        ],
        out_specs=pl.BlockSpec((BLK, D), lambda m, be: (m, 0)),
    )
    return pl.pallas_call(
        _k3_body,
        grid_spec=grid_spec,
        out_shape=jax.ShapeDtypeStruct((P, D), F32),
        compiler_params=pltpu.CompilerParams(
            dimension_semantics=("arbitrary",),
            vmem_limit_bytes=110 * 1024 * 1024),
    )(bexp, xs, wg, wu, wd)


def _k4_body(ys_hbm, pos0_hbm, pos1_hbm, y0_hbm, y1_hbm,
             rowbuf0, rowbuf1, idx0, idx1, sem0, sem1):
    wid = lax.axis_index("s") * 2 + lax.axis_index("c")
    pltpu.sync_copy(pos0_hbm.at[pl.ds(wid * (TPT // 16), TPT // 16)], idx0)
    pltpu.sync_copy(pos1_hbm.at[pl.ds(wid * (TPT // 16), TPT // 16)], idx1)
    base = wid * TPT
    for c in range(TPT // 16):
        d0 = pltpu.async_copy(ys_hbm.at[idx0.at[c]], rowbuf0, sem0)
        d1 = pltpu.async_copy(ys_hbm.at[idx1.at[c]], rowbuf1, sem1)
        d0.wait()
        pltpu.sync_copy(rowbuf0, y0_hbm.at[pl.ds(base + c * 16, 16)])
        d1.wait()
        pltpu.sync_copy(rowbuf1, y1_hbm.at[pl.ds(base + c * 16, 16)])


def _run_k4(ys, pos0, pos1):
    mesh = plsc.VectorSubcoreMesh(core_axis_name="c", subcore_axis_name="s")
    return pl.kernel(
        _k4_body,
        out_type=[jax.ShapeDtypeStruct((T, D), F32),
                  jax.ShapeDtypeStruct((T, D), F32)],
        mesh=mesh,
        scratch_types=[
            pltpu.VMEM((16, D), F32),
            pltpu.VMEM((16, D), F32),
            pltpu.VMEM((TPT // 16, 16), I32),
            pltpu.VMEM((TPT // 16, 16), I32),
            pltpu.SemaphoreType.DMA,
            pltpu.SemaphoreType.DMA,
        ],
    )(ys, pos0.reshape(T // 16, 16), pos1.reshape(T // 16, 16))


K5B = 512
K5N = T // K5B


def _k5_body(y0_ref, y1_ref, rw0_ref, rw1_ref, sel0_ref, sel1_ref,
             final_ref, cos_ref, l2o_ref, gacc, sacc):
    m = pl.program_id(0)

    @pl.when(m == 0)
    def _():
        gacc[...] = jnp.zeros((E, E), F32)
        sacc[...] = jnp.zeros((1, E), F32)

    rw0 = rw0_ref[...].reshape(K5B, 1)
    rw1 = rw1_ref[...].reshape(K5B, 1)
    y0 = y0_ref[...] * rw0
    y1 = y1_ref[...] * rw1
    final_ref[...] = y0 + y1

    p = jnp.sum(y0 * y1, axis=1, keepdims=True)
    d0 = jnp.sum(y0 * y0, axis=1, keepdims=True)
    d1 = jnp.sum(y1 * y1, axis=1, keepdims=True)
    s0 = jnp.sum(y0, axis=1, keepdims=True)
    s1 = jnp.sum(y1, axis=1, keepdims=True)
    iot = lax.broadcasted_iota(I32, (K5B, E), 1)
    oh0 = (sel0_ref[...].reshape(K5B, 1) == iot).astype(F32)
    oh1 = (sel1_ref[...].reshape(K5B, 1) == iot).astype(F32)

    dn = (((0,), (0,)), ((), ()))
    hi = lax.Precision.HIGHEST
    spart = (lax.dot_general(s0, oh0, dn, precision=hi)
             + lax.dot_general(s1, oh1, dn, precision=hi))
    dpart = (lax.dot_general(d0, oh0, dn, precision=hi)
             + lax.dot_general(d1, oh1, dn, precision=hi))
    m01 = lax.dot_general(oh0, p * oh1, dn, precision=hi)
    m10 = lax.dot_general(oh1, p * oh0, dn, precision=hi)
    eye = (lax.broadcasted_iota(I32, (E, E), 0)
           == lax.broadcasted_iota(I32, (E, E), 1)).astype(F32)
    gacc[...] += m01 + m10 + eye * dpart
    sacc[...] += spart

    @pl.when(m == K5N - 1)
    def _():
        n_tot = float(T * D)
        g = gacc[...]
        s = sacc[...]
        c = g - lax.dot_general(s, s, dn, precision=hi) / n_tot
        cd = jnp.sum(c * eye, axis=0, keepdims=True)
        den = jnp.maximum(jnp.sqrt(jnp.maximum(cd, 0.0)), 1e-12)
        denom2 = lax.dot_general(den, den, dn, precision=hi)
        cos = c / denom2
        cos_ref[...] = cos
        sqn = jnp.sum(cos * eye, axis=0, keepdims=True)
        sq_i = lax.dot_general(eye, sqn, (((1,), (1,)), ((), ())),
                               precision=hi)
        l2o_ref[...] = jnp.sqrt(jnp.maximum(sq_i + sqn - 2.0 * cos, 0.0))


def _run_k5(y0, y1, rw, sel):
    rw0 = rw[:, 0].reshape(K5N, K5B, 1)
    rw1 = rw[:, 1].reshape(K5N, K5B, 1)
    sel0 = sel[:, 0].reshape(K5N, K5B, 1)
    sel1 = sel[:, 1].reshape(K5N, K5B, 1)
    return pl.pallas_call(
        _k5_body,
        grid=(K5N,),
        in_specs=[
            pl.BlockSpec((K5B, D), lambda m: (m, 0)),
            pl.BlockSpec((K5B, D), lambda m: (m, 0)),
            pl.BlockSpec((1, K5B, 1), lambda m: (m, 0, 0)),
            pl.BlockSpec((1, K5B, 1), lambda m: (m, 0, 0)),
            pl.BlockSpec((1, K5B, 1), lambda m: (m, 0, 0)),
            pl.BlockSpec((1, K5B, 1), lambda m: (m, 0, 0)),
        ],
        out_specs=[
            pl.BlockSpec((K5B, D), lambda m: (m, 0)),
            pl.BlockSpec((E, E), lambda m: (0, 0)),
            pl.BlockSpec((E, E), lambda m: (0, 0)),
        ],
        out_shape=[
            jax.ShapeDtypeStruct((T, D), F32),
            jax.ShapeDtypeStruct((E, E), F32),
            jax.ShapeDtypeStruct((E, E), F32),
        ],
        scratch_shapes=[pltpu.VMEM((E, E), F32), pltpu.VMEM((1, E), F32)],
        compiler_params=pltpu.CompilerParams(
            dimension_semantics=("arbitrary",),
            vmem_limit_bytes=100 * 1024 * 1024),
    )(y0, y1, rw0, rw1, sel0, sel1)


def kernel(hidden_states, W_router, Wg, Wu, Wd):
    b, s, d = hidden_states.shape
    x = hidden_states.reshape(-1, d)
    (logits, cosine_logits, l2_logits, pos, rw, sel, bexp) = _run_k1(
        x, W_router)
    pos0 = pos[:, 0]
    pos1 = pos[:, 1]
    xs = _run_k2(x, pos0, pos1)
    ys = _run_k3(xs, bexp.reshape(NBLK), Wg, Wu, Wd)
    y0, y1 = _run_k4(ys, pos0, pos1)
    final, cosine_out, l2_out = _run_k5(y0, y1, rw, sel)
    return (final.reshape(b, s, d), logits, cosine_logits, l2_logits,
            cosine_out, l2_out)
